# baseline (speedup 1.0000x reference)
import jax
import jax.numpy as jnp
from jax import lax
from jax.experimental import pallas as pl
from jax.experimental.pallas import tpu as pltpu
from jax.experimental.pallas import tpu_sc as plsc

N = 10000
E = 320000
D = 128
NT = 16
K = 64
CB = 32
NB = 10
NBUF = 4
PF = 3
EPT = NB * CB * K
E_PAD = NT * EPT
N_PAD = 10240
RPT = N_PAD // NT


def _sc_body(xs, src_both, dst_both, y1, y2,
             acc, sidx, didx, rows, gsems, ssems, isems):
    c = lax.axis_index("c")
    s = lax.axis_index("s")
    src_hbm = src_both.at[c]
    dst_hbm = dst_both.at[c]

    def idx_load(n, ib):
        blk = pl.ds(n * CB, CB)
        pltpu.async_copy(src_hbm.at[s, blk], sidx.at[ib], isems[ib])
        pltpu.async_copy(dst_hbm.at[s, blk], didx.at[ib], isems[ib])

    def idx_wait(ib):
        blk0 = pl.ds(0, CB)
        pltpu.make_async_copy(src_hbm.at[s, blk0], sidx.at[ib], isems[ib]).wait()
        pltpu.make_async_copy(src_hbm.at[s, blk0], didx.at[ib], isems[ib]).wait()

    def hop(x_hbm, y_hbm):
        r0 = rows.at[0]

        @pl.loop(0, K)
        def _zfill(i):
            for g8 in range(8):
                r0[i, pl.ds(g8 * 16, 16)] = jnp.zeros((16,), jnp.float32)

        for k in range(RPT // K):
            pltpu.sync_copy(r0, acc.at[pl.ds(s * RPT + k * K, K)])
        plsc.subcore_barrier()

        def gwait(b):
            pltpu.make_async_copy(x_hbm.at[pl.ds(0, K)], rows.at[b],
                                  gsems[b]).wait()

        def swait(b):
            pltpu.make_async_copy(rows.at[b], acc.at[pl.ds(0, K)],
                                  ssems[b]).wait()

        idx_load(0, 0)

        @pl.loop(0, NB // 2)
        def _blocks(m):
            for ib in (0, 1):
                n = m * 2 + ib
                idx_wait(ib)
                @pl.when(n >= 1)
                def _():
                    for b in range(PF, NBUF):
                        swait(b)

                @pl.when(n + 1 < NB)
                def _():
                    idx_load(n + 1, 1 - ib)

                si = sidx.at[ib]
                di = didx.at[ib]
                for b in range(PF):
                    pltpu.async_copy(x_hbm.at[si.at[b]], rows.at[b], gsems[b])

                @pl.loop(0, CB // NBUF)
                def _ring(q):
                    for b in range(NBUF):
                        t = q * NBUF + b
                        bn = (b + PF) % NBUF

                        @pl.when(t >= NBUF - PF)
                        def _():
                            swait(bn)

                        @pl.when(t + PF < CB)
                        def _():
                            pltpu.async_copy(
                                x_hbm.at[si.at[t + PF]], rows.at[bn], gsems[bn])

                        gwait(b)
                        pltpu.async_copy(rows.at[b], acc.at[di.at[t]],
                                         ssems[b], add=True)

        for b in range(PF, NBUF):
            swait(b)
        plsc.subcore_barrier()
        pltpu.sync_copy(acc.at[pl.ds(s * RPT, RPT)], y_hbm.at[pl.ds(s * RPT, RPT)])
        plsc.subcore_barrier()

    hop(xs.at[c], y1.at[c])
    hop(y1.at[c], y2.at[c])


_SC_SPMM_CACHE = []


def _sc_spmm_call():
    if not _SC_SPMM_CACHE:
        _SC_SPMM_CACHE.append(pl.kernel(
            _sc_body,
            out_type=[jax.ShapeDtypeStruct((2, N_PAD, D), jnp.float32)] * 2,
            mesh=plsc.VectorSubcoreMesh(core_axis_name="c", subcore_axis_name="s"),
            scratch_types=[
                pltpu.VMEM_SHARED((N_PAD, D), jnp.float32),
                pltpu.VMEM((2, CB, K), jnp.int32),
                pltpu.VMEM((2, CB, K), jnp.int32),
                pltpu.VMEM((NBUF, K, D), jnp.float32),
                [pltpu.SemaphoreType.DMA] * NBUF,
                [pltpu.SemaphoreType.DMA] * NBUF,
                [pltpu.SemaphoreType.DMA] * 2,
            ],
        ))
    return _SC_SPMM_CACHE[0]


def _scale_body(h_ref, dg_ref, x_ref):
    x_ref[...] = (h_ref[...] * dg_ref[0])[None]


_BS = 1000

_scale = pl.pallas_call(
    _scale_body,
    grid=(N // _BS, 2),
    in_specs=[
        pl.BlockSpec((_BS, D), lambda i, j: (i, 0)),
        pl.BlockSpec((1, _BS, 1), lambda i, j: (j, i, 0)),
    ],
    out_specs=pl.BlockSpec((1, _BS, D), lambda i, j: (j, i, 0)),
    out_shape=jax.ShapeDtypeStruct((2, N, D), jnp.float32),
)


def _combine_body(coef_ref, yo1, yi1, yo2, yi2, h_ref, th_ref, o_ref):
    s = (coef_ref[0] * yo1[0] + coef_ref[1] * yi1[0]
         + coef_ref[2] * yo2[0] + coef_ref[3] * yi2[0])
    z = jnp.dot(s, th_ref[...], preferred_element_type=jnp.float32)
    o_ref[...] = 1.0 / (1.0 + jnp.exp(-z)) + h_ref[...]


_combine = pl.pallas_call(
    _combine_body,
    grid=(N // _BS,),
    in_specs=[
        pl.BlockSpec(memory_space=pltpu.SMEM),
        pl.BlockSpec((1, _BS, D), lambda i: (0, i, 0)),
        pl.BlockSpec((1, _BS, D), lambda i: (1, i, 0)),
        pl.BlockSpec((1, _BS, D), lambda i: (0, i, 0)),
        pl.BlockSpec((1, _BS, D), lambda i: (1, i, 0)),
        pl.BlockSpec((_BS, D), lambda i: (i, 0)),
        pl.BlockSpec((D, D), lambda i: (0, 0)),
    ],
    out_specs=pl.BlockSpec((_BS, D), lambda i: (i, 0)),
    out_shape=jax.ShapeDtypeStruct((N, D), jnp.float32),
)


def kernel(H_l, hop_attention, theta_out, theta_in, Theta, out_degree,
           in_degree, edge_weight, edge_index):
    row = edge_index[0]
    col = edge_index[1]
    pad = E_PAD - E
    pad_sink = jnp.full((pad,), N, jnp.int32)
    pad_zero = jnp.zeros((pad,), jnp.int32)
    src_col = jnp.concatenate([col, pad_zero]).reshape(NT, NB * CB, K)
    dst_row = jnp.concatenate([row, pad_sink]).reshape(NT, NB * CB, K)
    src_row = jnp.concatenate([row, pad_zero]).reshape(NT, NB * CB, K)
    dst_col = jnp.concatenate([col, pad_sink]).reshape(NT, NB * CB, K)
    src_both = jnp.stack([src_col, src_row])
    dst_both = jnp.stack([dst_row, dst_col])

    degs = jnp.stack([out_degree, in_degree])[:, :, None]
    xs = _scale(H_l, degs)

    y1, y2 = _sc_spmm_call()(xs, src_both, dst_both)

    alpha = jax.nn.softmax(hop_attention, axis=0)
    coef = jnp.stack([
        alpha[0] * theta_out[0], alpha[0] * theta_in[0],
        alpha[1] * theta_out[1], alpha[1] * theta_in[1],
    ])

    return _combine(coef, y1, y1, y2, y2, H_l, Theta)


# speedup vs baseline: 1.1128x; 1.1128x over previous
import jax
import jax.numpy as jnp
from jax import lax
from jax.experimental import pallas as pl
from jax.experimental.pallas import tpu as pltpu
from jax.experimental.pallas import tpu_sc as plsc

N = 10000
E = 320000
D = 128
NT = 16
K = 64
CB = 32
NB = 10
NBUF = 4
PF = 3
EPT = NB * CB * K
E_PAD = NT * EPT
N_PAD = 10240
RPT = N_PAD // NT


def _sc_body(xo, xi, src_col, dst_row, src_row, dst_col,
             yo1, yo2, yi1, yi2,
             acc, sidx, didx, rows, gsems, ssems, isems):
    c = lax.axis_index("c")
    s = lax.axis_index("s")

    def idx_load(src_hbm, dst_hbm, n, ib):
        blk = pl.ds(n * CB, CB)
        pltpu.async_copy(src_hbm.at[s, blk], sidx.at[ib], isems[ib])
        pltpu.async_copy(dst_hbm.at[s, blk], didx.at[ib], isems[ib])

    def idx_wait(src_hbm, ib):
        blk0 = pl.ds(0, CB)
        pltpu.make_async_copy(src_hbm.at[s, blk0], sidx.at[ib], isems[ib]).wait()
        pltpu.make_async_copy(src_hbm.at[s, blk0], didx.at[ib], isems[ib]).wait()

    def hop(src_hbm, dst_hbm, x_hbm, y_hbm):
        r0 = rows.at[0]

        @pl.loop(0, K)
        def _zfill(i):
            for g8 in range(8):
                r0[i, pl.ds(g8 * 16, 16)] = jnp.zeros((16,), jnp.float32)

        for k in range(RPT // K):
            pltpu.sync_copy(r0, acc.at[pl.ds(s * RPT + k * K, K)])
        plsc.subcore_barrier()

        def gwait(b):
            pltpu.make_async_copy(x_hbm.at[pl.ds(0, K)], rows.at[b],
                                  gsems[b]).wait()

        def swait(b):
            pltpu.make_async_copy(rows.at[b], acc.at[pl.ds(0, K)],
                                  ssems[b]).wait()

        idx_load(src_hbm, dst_hbm, 0, 0)

        @pl.loop(0, NB // 2)
        def _blocks(m):
            for ib in (0, 1):
                n = m * 2 + ib
                idx_wait(src_hbm, ib)
                @pl.when(n >= 1)
                def _():
                    for b in range(PF, NBUF):
                        swait(b)

                @pl.when(n + 1 < NB)
                def _():
                    idx_load(src_hbm, dst_hbm, n + 1, 1 - ib)

                si = sidx.at[ib]
                di = didx.at[ib]
                for b in range(PF):
                    pltpu.async_copy(x_hbm.at[si.at[b]], rows.at[b], gsems[b])

                @pl.loop(0, CB // NBUF)
                def _ring(q):
                    for b in range(NBUF):
                        t = q * NBUF + b
                        bn = (b + PF) % NBUF

                        @pl.when(t >= NBUF - PF)
                        def _():
                            swait(bn)

                        @pl.when(t + PF < CB)
                        def _():
                            pltpu.async_copy(
                                x_hbm.at[si.at[t + PF]], rows.at[bn], gsems[bn])

                        gwait(b)
                        pltpu.async_copy(rows.at[b], acc.at[di.at[t]],
                                         ssems[b], add=True)

        for b in range(PF, NBUF):
            swait(b)
        plsc.subcore_barrier()
        pltpu.sync_copy(acc.at[pl.ds(s * RPT, RPT)], y_hbm.at[pl.ds(s * RPT, RPT)])

    @pl.when(c == 0)
    def _():
        hop(src_col, dst_row, xo, yo1)
        hop(src_col, dst_row, yo1, yo2)

    @pl.when(c == 1)
    def _():
        hop(src_row, dst_col, xi, yi1)
        hop(src_row, dst_col, yi1, yi2)


_SC_SPMM_CACHE = []


def _sc_spmm_call():
    if not _SC_SPMM_CACHE:
        _SC_SPMM_CACHE.append(pl.kernel(
            _sc_body,
            out_type=[jax.ShapeDtypeStruct((N_PAD, D), jnp.float32)] * 4,
            mesh=plsc.VectorSubcoreMesh(core_axis_name="c", subcore_axis_name="s"),
            scratch_types=[
                pltpu.VMEM_SHARED((N_PAD, D), jnp.float32),
                pltpu.VMEM((2, CB, K), jnp.int32),
                pltpu.VMEM((2, CB, K), jnp.int32),
                pltpu.VMEM((NBUF, K, D), jnp.float32),
                [pltpu.SemaphoreType.DMA] * NBUF,
                [pltpu.SemaphoreType.DMA] * NBUF,
                [pltpu.SemaphoreType.DMA] * 2,
            ],
        ))
    return _SC_SPMM_CACHE[0]


def _scale_body(h_ref, do_ref, di_ref, xo_ref, xi_ref):
    h = h_ref[...]
    xo_ref[...] = h * do_ref[...]
    xi_ref[...] = h * di_ref[...]


_BS = 1000

_scale = pl.pallas_call(
    _scale_body,
    grid=(N // _BS,),
    in_specs=[
        pl.BlockSpec((_BS, D), lambda i: (i, 0)),
        pl.BlockSpec((_BS, 1), lambda i: (i, 0)),
        pl.BlockSpec((_BS, 1), lambda i: (i, 0)),
    ],
    out_specs=[pl.BlockSpec((_BS, D), lambda i: (i, 0))] * 2,
    out_shape=[jax.ShapeDtypeStruct((N, D), jnp.float32)] * 2,
)


def _combine_body(coef_ref, yo1, yi1, yo2, yi2, h_ref, th_ref, o_ref):
    s = (coef_ref[0] * yo1[...] + coef_ref[1] * yi1[...]
         + coef_ref[2] * yo2[...] + coef_ref[3] * yi2[...])
    z = jnp.dot(s, th_ref[...], preferred_element_type=jnp.float32)
    o_ref[...] = 1.0 / (1.0 + jnp.exp(-z)) + h_ref[...]


_combine = pl.pallas_call(
    _combine_body,
    grid=(N // _BS,),
    in_specs=[
        pl.BlockSpec(memory_space=pltpu.SMEM),
        pl.BlockSpec((_BS, D), lambda i: (i, 0)),
        pl.BlockSpec((_BS, D), lambda i: (i, 0)),
        pl.BlockSpec((_BS, D), lambda i: (i, 0)),
        pl.BlockSpec((_BS, D), lambda i: (i, 0)),
        pl.BlockSpec((_BS, D), lambda i: (i, 0)),
        pl.BlockSpec((D, D), lambda i: (0, 0)),
    ],
    out_specs=pl.BlockSpec((_BS, D), lambda i: (i, 0)),
    out_shape=jax.ShapeDtypeStruct((N, D), jnp.float32),
)


def kernel(H_l, hop_attention, theta_out, theta_in, Theta, out_degree,
           in_degree, edge_weight, edge_index):
    row = edge_index[0]
    col = edge_index[1]
    pad = E_PAD - E
    pad_sink = jnp.full((pad,), N, jnp.int32)
    pad_zero = jnp.zeros((pad,), jnp.int32)
    src_col = jnp.concatenate([col, pad_zero]).reshape(NT, NB * CB, K)
    dst_row = jnp.concatenate([row, pad_sink]).reshape(NT, NB * CB, K)
    src_row = jnp.concatenate([row, pad_zero]).reshape(NT, NB * CB, K)
    dst_col = jnp.concatenate([col, pad_sink]).reshape(NT, NB * CB, K)

    xo, xi = _scale(H_l, out_degree[:, None], in_degree[:, None])

    yo1, yo2, yi1, yi2 = _sc_spmm_call()(xo, xi, src_col, dst_row, src_row, dst_col)

    alpha = jax.nn.softmax(hop_attention, axis=0)
    coef = jnp.stack([
        alpha[0] * theta_out[0], alpha[0] * theta_in[0],
        alpha[1] * theta_out[1], alpha[1] * theta_in[1],
    ])

    return _combine(coef, yo1, yi1, yo2, yi2, H_l, Theta)


# speedup vs baseline: 1.1163x; 1.0031x over previous
import jax
import jax.numpy as jnp
from jax import lax
from jax.experimental import pallas as pl
from jax.experimental.pallas import tpu as pltpu
from jax.experimental.pallas import tpu_sc as plsc

N = 10000
E = 320000
D = 128
NT = 16
K = 64
CB = 32
NB = 10
NBUF = 4
PF = 3
EPT = NB * CB * K
E_PAD = NT * EPT
N_PAD = 10240
RPT = N_PAD // NT


def _sc_body(xo, xi, src_col, dst_row, src_row, dst_col,
             yo1, yo2, yi1, yi2,
             acc, sidx, didx, rows, gsems, ssems, isems):
    c = lax.axis_index("c")
    s = lax.axis_index("s")

    def idx_load(src_hbm, dst_hbm, n, ib):
        blk = pl.ds(n * CB, CB)
        pltpu.async_copy(src_hbm.at[s, blk], sidx.at[ib], isems[ib])
        pltpu.async_copy(dst_hbm.at[s, blk], didx.at[ib], isems[ib])

    def idx_wait(src_hbm, ib):
        blk0 = pl.ds(0, CB)
        pltpu.make_async_copy(src_hbm.at[s, blk0], sidx.at[ib], isems[ib]).wait()
        pltpu.make_async_copy(src_hbm.at[s, blk0], didx.at[ib], isems[ib]).wait()

    def hop(src_hbm, dst_hbm, x_hbm, y_hbm):
        r0 = rows.at[0]

        @pl.loop(0, K)
        def _zfill(i):
            for g8 in range(8):
                r0[i, pl.ds(g8 * 16, 16)] = jnp.zeros((16,), jnp.float32)

        idx_load(src_hbm, dst_hbm, 0, 0)
        for k in range(RPT // K):
            pltpu.async_copy(r0, acc.at[pl.ds(s * RPT + k * K, K)], gsems[0])
        for k in range(RPT // K):
            pltpu.make_async_copy(r0, acc.at[pl.ds(0, K)], gsems[0]).wait()
        plsc.subcore_barrier()

        def gwait(b):
            pltpu.make_async_copy(x_hbm.at[pl.ds(0, K)], rows.at[b],
                                  gsems[b]).wait()

        def swait(b):
            pltpu.make_async_copy(rows.at[b], acc.at[pl.ds(0, K)],
                                  ssems[b]).wait()

        @pl.loop(0, NB // 2)
        def _blocks(m):
            for ib in (0, 1):
                n = m * 2 + ib
                idx_wait(src_hbm, ib)
                @pl.when(n >= 1)
                def _():
                    for b in range(PF, NBUF):
                        swait(b)

                @pl.when(n + 1 < NB)
                def _():
                    idx_load(src_hbm, dst_hbm, n + 1, 1 - ib)

                si = sidx.at[ib]
                di = didx.at[ib]
                for b in range(PF):
                    pltpu.async_copy(x_hbm.at[si.at[b]], rows.at[b], gsems[b])

                @pl.loop(0, CB // NBUF)
                def _ring(q):
                    for b in range(NBUF):
                        t = q * NBUF + b
                        bn = (b + PF) % NBUF

                        @pl.when(t >= NBUF - PF)
                        def _():
                            swait(bn)

                        @pl.when(t + PF < CB)
                        def _():
                            pltpu.async_copy(
                                x_hbm.at[si.at[t + PF]], rows.at[bn], gsems[bn])

                        gwait(b)
                        pltpu.async_copy(rows.at[b], acc.at[di.at[t]],
                                         ssems[b], add=True)

        for b in range(PF, NBUF):
            swait(b)
        plsc.subcore_barrier()
        pltpu.sync_copy(acc.at[pl.ds(s * RPT, RPT)], y_hbm.at[pl.ds(s * RPT, RPT)])

    @pl.when(c == 0)
    def _():
        hop(src_col, dst_row, xo, yo1)
        hop(src_col, dst_row, yo1, yo2)

    @pl.when(c == 1)
    def _():
        hop(src_row, dst_col, xi, yi1)
        hop(src_row, dst_col, yi1, yi2)


_SC_SPMM_CACHE = []


def _sc_spmm_call():
    if not _SC_SPMM_CACHE:
        _SC_SPMM_CACHE.append(pl.kernel(
            _sc_body,
            out_type=[jax.ShapeDtypeStruct((N_PAD, D), jnp.float32)] * 4,
            mesh=plsc.VectorSubcoreMesh(core_axis_name="c", subcore_axis_name="s"),
            scratch_types=[
                pltpu.VMEM_SHARED((N_PAD, D), jnp.float32),
                pltpu.VMEM((2, CB, K), jnp.int32),
                pltpu.VMEM((2, CB, K), jnp.int32),
                pltpu.VMEM((NBUF, K, D), jnp.float32),
                [pltpu.SemaphoreType.DMA] * NBUF,
                [pltpu.SemaphoreType.DMA] * NBUF,
                [pltpu.SemaphoreType.DMA] * 2,
            ],
        ))
    return _SC_SPMM_CACHE[0]


def _scale_body(h_ref, do_ref, di_ref, xo_ref, xi_ref):
    h = h_ref[...]
    xo_ref[...] = h * do_ref[...]
    xi_ref[...] = h * di_ref[...]


_BS = 1000

_scale = pl.pallas_call(
    _scale_body,
    grid=(N // _BS,),
    in_specs=[
        pl.BlockSpec((_BS, D), lambda i: (i, 0)),
        pl.BlockSpec((_BS, 1), lambda i: (i, 0)),
        pl.BlockSpec((_BS, 1), lambda i: (i, 0)),
    ],
    out_specs=[pl.BlockSpec((_BS, D), lambda i: (i, 0))] * 2,
    out_shape=[jax.ShapeDtypeStruct((N, D), jnp.float32)] * 2,
)


def _combine_body(coef_ref, yo1, yi1, yo2, yi2, h_ref, th_ref, o_ref):
    s = (coef_ref[0] * yo1[...] + coef_ref[1] * yi1[...]
         + coef_ref[2] * yo2[...] + coef_ref[3] * yi2[...])
    z = jnp.dot(s, th_ref[...], preferred_element_type=jnp.float32)
    o_ref[...] = 1.0 / (1.0 + jnp.exp(-z)) + h_ref[...]


_combine = pl.pallas_call(
    _combine_body,
    grid=(N // _BS,),
    in_specs=[
        pl.BlockSpec(memory_space=pltpu.SMEM),
        pl.BlockSpec((_BS, D), lambda i: (i, 0)),
        pl.BlockSpec((_BS, D), lambda i: (i, 0)),
        pl.BlockSpec((_BS, D), lambda i: (i, 0)),
        pl.BlockSpec((_BS, D), lambda i: (i, 0)),
        pl.BlockSpec((_BS, D), lambda i: (i, 0)),
        pl.BlockSpec((D, D), lambda i: (0, 0)),
    ],
    out_specs=pl.BlockSpec((_BS, D), lambda i: (i, 0)),
    out_shape=jax.ShapeDtypeStruct((N, D), jnp.float32),
)


def kernel(H_l, hop_attention, theta_out, theta_in, Theta, out_degree,
           in_degree, edge_weight, edge_index):
    row = edge_index[0]
    col = edge_index[1]
    pad = E_PAD - E
    pad_sink = jnp.full((pad,), N, jnp.int32)
    pad_zero = jnp.zeros((pad,), jnp.int32)
    src_col = jnp.concatenate([col, pad_zero]).reshape(NT, NB * CB, K)
    dst_row = jnp.concatenate([row, pad_sink]).reshape(NT, NB * CB, K)
    src_row = jnp.concatenate([row, pad_zero]).reshape(NT, NB * CB, K)
    dst_col = jnp.concatenate([col, pad_sink]).reshape(NT, NB * CB, K)

    xo, xi = _scale(H_l, out_degree[:, None], in_degree[:, None])

    yo1, yo2, yi1, yi2 = _sc_spmm_call()(xo, xi, src_col, dst_row, src_row, dst_col)

    alpha = jax.nn.softmax(hop_attention, axis=0)
    coef = jnp.stack([
        alpha[0] * theta_out[0], alpha[0] * theta_in[0],
        alpha[1] * theta_out[1], alpha[1] * theta_in[1],
    ])

    return _combine(coef, yo1, yi1, yo2, yi2, H_l, Theta)


# speedup vs baseline: 1.1219x; 1.0050x over previous
import jax
import jax.numpy as jnp
from jax import lax
from jax.experimental import pallas as pl
from jax.experimental.pallas import tpu as pltpu
from jax.experimental.pallas import tpu_sc as plsc

N = 10000
E = 320000
D = 128
NT = 16
K = 64
CB = 32
NB = 10
NBUF = 4
PF = 3
EPT = NB * CB * K
E_PAD = NT * EPT
N_PAD = 10240
RPT = N_PAD // NT


def _sc_body(xo, xi, src_col, dst_row, src_row, dst_col,
             yo1, yo2, yi1, yi2,
             acc, sidx, didx, rows, gsems, ssems, isems):
    c = lax.axis_index("c")
    s = lax.axis_index("s")

    def idx_load(src_hbm, dst_hbm, n, ib):
        blk = pl.ds(n * CB, CB)
        pltpu.async_copy(src_hbm.at[s, blk], sidx.at[ib], isems[ib])
        pltpu.async_copy(dst_hbm.at[s, blk], didx.at[ib], isems[ib])

    def idx_wait(src_hbm, ib):
        blk0 = pl.ds(0, CB)
        pltpu.make_async_copy(src_hbm.at[s, blk0], sidx.at[ib], isems[ib]).wait()
        pltpu.make_async_copy(src_hbm.at[s, blk0], didx.at[ib], isems[ib]).wait()

    def hop(src_hbm, dst_hbm, x_hbm, y_hbm):
        r0 = rows.at[0]

        @pl.loop(0, K)
        def _zfill(i):
            for g8 in range(8):
                r0[i, pl.ds(g8 * 16, 16)] = jnp.zeros((16,), jnp.float32)

        idx_load(src_hbm, dst_hbm, 0, 0)
        for k in range(RPT // K):
            pltpu.async_copy(r0, acc.at[pl.ds(s * RPT + k * K, K)], gsems[0])
        for k in range(RPT // K):
            pltpu.make_async_copy(r0, acc.at[pl.ds(0, K)], gsems[0]).wait()
        plsc.subcore_barrier()

        def gwait(b):
            pltpu.make_async_copy(x_hbm.at[pl.ds(0, K)], rows.at[b],
                                  gsems[b]).wait()

        def swait(b):
            pltpu.make_async_copy(rows.at[b], acc.at[pl.ds(0, K)],
                                  ssems[b]).wait()

        idx_wait(src_hbm, 0)
        si0 = sidx.at[0]
        for b in range(PF):
            pltpu.async_copy(x_hbm.at[si0.at[b]], rows.at[b], gsems[b])

        @pl.loop(0, NB // 2)
        def _blocks(m):
            for ib in (0, 1):
                n = m * 2 + ib
                @pl.when(n >= 1)
                def _():
                    for b in range(PF, NBUF):
                        swait(b)

                @pl.when(n + 1 < NB)
                def _():
                    idx_load(src_hbm, dst_hbm, n + 1, 1 - ib)

                si = sidx.at[ib]
                di = didx.at[ib]
                @pl.loop(0, CB // NBUF)
                def _ring(q):
                    for b in range(NBUF):
                        t = q * NBUF + b
                        bn = (b + PF) % NBUF

                        @pl.when(t >= NBUF - PF)
                        def _():
                            swait(bn)

                        @pl.when(t + PF < CB)
                        def _():
                            pltpu.async_copy(
                                x_hbm.at[si.at[t + PF]], rows.at[bn], gsems[bn])

                        gwait(b)
                        pltpu.async_copy(rows.at[b], acc.at[di.at[t]],
                                         ssems[b], add=True)

                @pl.when(n + 1 < NB)
                def _():
                    idx_wait(src_hbm, 1 - ib)
                    sj = sidx.at[1 - ib]
                    for b in range(PF):
                        pltpu.async_copy(x_hbm.at[sj.at[b]], rows.at[b],
                                         gsems[b])

        for b in range(PF, NBUF):
            swait(b)
        plsc.subcore_barrier()
        pltpu.sync_copy(acc.at[pl.ds(s * RPT, RPT)], y_hbm.at[pl.ds(s * RPT, RPT)])

    @pl.when(c == 0)
    def _():
        hop(src_col, dst_row, xo, yo1)
        hop(src_col, dst_row, yo1, yo2)

    @pl.when(c == 1)
    def _():
        hop(src_row, dst_col, xi, yi1)
        hop(src_row, dst_col, yi1, yi2)


_SC_SPMM_CACHE = []


def _sc_spmm_call():
    if not _SC_SPMM_CACHE:
        _SC_SPMM_CACHE.append(pl.kernel(
            _sc_body,
            out_type=[jax.ShapeDtypeStruct((N_PAD, D), jnp.float32)] * 4,
            mesh=plsc.VectorSubcoreMesh(core_axis_name="c", subcore_axis_name="s"),
            scratch_types=[
                pltpu.VMEM_SHARED((N_PAD, D), jnp.float32),
                pltpu.VMEM((2, CB, K), jnp.int32),
                pltpu.VMEM((2, CB, K), jnp.int32),
                pltpu.VMEM((NBUF, K, D), jnp.float32),
                [pltpu.SemaphoreType.DMA] * NBUF,
                [pltpu.SemaphoreType.DMA] * NBUF,
                [pltpu.SemaphoreType.DMA] * 2,
            ],
        ))
    return _SC_SPMM_CACHE[0]


def _scale_body(h_ref, do_ref, di_ref, xo_ref, xi_ref):
    h = h_ref[...]
    xo_ref[...] = h * do_ref[...]
    xi_ref[...] = h * di_ref[...]


_BS = 1000

_scale = pl.pallas_call(
    _scale_body,
    grid=(N // _BS,),
    in_specs=[
        pl.BlockSpec((_BS, D), lambda i: (i, 0)),
        pl.BlockSpec((_BS, 1), lambda i: (i, 0)),
        pl.BlockSpec((_BS, 1), lambda i: (i, 0)),
    ],
    out_specs=[pl.BlockSpec((_BS, D), lambda i: (i, 0))] * 2,
    out_shape=[jax.ShapeDtypeStruct((N, D), jnp.float32)] * 2,
)


def _combine_body(coef_ref, yo1, yi1, yo2, yi2, h_ref, th_ref, o_ref):
    s = (coef_ref[0] * yo1[...] + coef_ref[1] * yi1[...]
         + coef_ref[2] * yo2[...] + coef_ref[3] * yi2[...])
    z = jnp.dot(s, th_ref[...], preferred_element_type=jnp.float32)
    o_ref[...] = 1.0 / (1.0 + jnp.exp(-z)) + h_ref[...]


_combine = pl.pallas_call(
    _combine_body,
    grid=(N // _BS,),
    in_specs=[
        pl.BlockSpec(memory_space=pltpu.SMEM),
        pl.BlockSpec((_BS, D), lambda i: (i, 0)),
        pl.BlockSpec((_BS, D), lambda i: (i, 0)),
        pl.BlockSpec((_BS, D), lambda i: (i, 0)),
        pl.BlockSpec((_BS, D), lambda i: (i, 0)),
        pl.BlockSpec((_BS, D), lambda i: (i, 0)),
        pl.BlockSpec((D, D), lambda i: (0, 0)),
    ],
    out_specs=pl.BlockSpec((_BS, D), lambda i: (i, 0)),
    out_shape=jax.ShapeDtypeStruct((N, D), jnp.float32),
)


def kernel(H_l, hop_attention, theta_out, theta_in, Theta, out_degree,
           in_degree, edge_weight, edge_index):
    row = edge_index[0]
    col = edge_index[1]
    pad = E_PAD - E
    pad_sink = jnp.full((pad,), N, jnp.int32)
    pad_zero = jnp.zeros((pad,), jnp.int32)
    src_col = jnp.concatenate([col, pad_zero]).reshape(NT, NB * CB, K)
    dst_row = jnp.concatenate([row, pad_sink]).reshape(NT, NB * CB, K)
    src_row = jnp.concatenate([row, pad_zero]).reshape(NT, NB * CB, K)
    dst_col = jnp.concatenate([col, pad_sink]).reshape(NT, NB * CB, K)

    xo, xi = _scale(H_l, out_degree[:, None], in_degree[:, None])

    yo1, yo2, yi1, yi2 = _sc_spmm_call()(xo, xi, src_col, dst_row, src_row, dst_col)

    alpha = jax.nn.softmax(hop_attention, axis=0)
    coef = jnp.stack([
        alpha[0] * theta_out[0], alpha[0] * theta_in[0],
        alpha[1] * theta_out[1], alpha[1] * theta_in[1],
    ])

    return _combine(coef, yo1, yi1, yo2, yi2, H_l, Theta)
